# baseline (speedup 1.0000x reference)
import functools

import jax
import jax.numpy as jnp
from jax import lax
from jax.experimental import pallas as pl
from jax.experimental.pallas import tpu as pltpu
from jax.experimental.pallas import tpu_sc as plsc

N_ENV = 64
BUF = 4096
N_OBS = 128
N_ACT = 32
BATCH = 256
OUT_D = N_OBS + N_ACT + N_OBS + 3
DROW_PAD = 384
L = 16
NB = BATCH // L


def _build_sc_kernel(num_cores, num_subcores, half):
    n_workers = num_cores * num_subcores
    ne = N_ENV // 2
    epw = ne // n_workers
    mesh = plsc.VectorSubcoreMesh(core_axis_name="c", subcore_axis_name="s")
    f32 = jnp.float32
    i32 = jnp.int32

    @functools.partial(
        pl.kernel,
        out_type=[
            jax.ShapeDtypeStruct((ne * BATCH, N_OBS), f32),
            jax.ShapeDtypeStruct((ne * BATCH, N_OBS), f32),
            jax.ShapeDtypeStruct((8, ne * BATCH), f32),
            jax.ShapeDtypeStruct((ne * N_ACT, BATCH), f32),
        ],
        mesh=mesh,
        compiler_params=pltpu.CompilerParams(needs_layout_passes=False),
        scratch_types=[
            pltpu.VMEM((epw,), i32),
            pltpu.VMEM((1,), i32),
            pltpu.VMEM((4,), i32),
            pltpu.VMEM((4,), i32),
            pltpu.VMEM((epw, BATCH), i32),
            pltpu.VMEM((1, BUF), f32),
            pltpu.VMEM((1, BUF), i32),
            pltpu.VMEM((1, BUF), i32),
            pltpu.VMEM((epw, DROW_PAD), f32),
            pltpu.VMEM((128,), i32),
            pltpu.VMEM((128,), i32),
            pltpu.VMEM((BATCH, N_OBS), f32),
            pltpu.VMEM((BATCH, N_OBS), f32),
            pltpu.VMEM((8, BATCH), f32),
            pltpu.VMEM((4, BUF), f32),
            pltpu.VMEM((4, BUF), f32),
            pltpu.VMEM((N_ACT, BATCH), f32),
            pltpu.VMEM((L,), i32),
            pltpu.SemaphoreType.DMA,
            pltpu.SemaphoreType.DMA,
            pltpu.SemaphoreType.DMA,
            pltpu.SemaphoreType.DMA,
        ],
    )
    def k(obs_hbm, nobs_hbm, act_hbm, rew_hbm, dn_hbm, tr_hbm, data_hbm,
          tvec_hbm, sidx_hbm,
          o_obs, o_nobs, o_tail, o_act,
          eidx, eidx1, fidx_a, fidx_b, sidx2, rew1, dn1, tr1, data2,
          gidx_a, gidx_b, obs_stage, nobs_stage, tail_stage,
          arow_a, arow_b, aout, tv, sem, sem2, sem3, sem4):
        wid = lax.axis_index("s") * num_cores + lax.axis_index("c")
        lane = lax.iota(i32, L)
        zero = jnp.full((L,), 0, i32)
        e0 = wid * epw + half * ne
        plsc.store_scatter(eidx, [lane], e0 + lane, mask=lane < epw)
        stage = [
            pltpu.async_copy(sidx_hbm.at[eidx], sidx2, sem),
            pltpu.async_copy(data_hbm.at[eidx], data2, sem),
        ]
        pltpu.sync_copy(tvec_hbm, tv)
        tvec = tv[...]
        for c in stage:
            c.wait()

        outcp = []
        for j in range(epw):
            e = e0 + j
            ebase = e * BUF
            plsc.store_scatter(eidx1, [lane], (e0 + j) + zero, mask=lane < 1)
            scopies = [
                pltpu.async_copy(rew_hbm.at[eidx1], rew1, sem2),
                pltpu.async_copy(dn_hbm.at[eidx1], dn1, sem2),
                pltpu.async_copy(tr_hbm.at[eidx1], tr1, sem2),
            ]
            jv = jnp.full((L,), j, i32)

            for kk in range(NB // 2):
                s = pl.ds(kk * L, L)
                gidx_a[s] = sidx2[j, s] + ebase
            for kk in range(NB // 2):
                s = pl.ds(kk * L, L)
                gidx_b[s] = sidx2[j, pl.ds(128 + kk * L, L)] + ebase
            for c in outcp:
                c.wait()
            outcp = []
            copies = []
            for h, gi in enumerate((gidx_a, gidx_b)):
                rows = pl.ds(h * 128, 128)
                copies.append(pltpu.async_copy(
                    obs_hbm.at[gi], obs_stage.at[rows], sem))
                copies.append(pltpu.async_copy(
                    nobs_hbm.at[gi], nobs_stage.at[rows], sem))

            NR = 4
            fbase = e * N_ACT
            bufs = (arow_a, arow_b)
            fidxs = (fidx_a, fidx_b)
            plsc.store_scatter(fidx_a, [lane], fbase + lane, mask=lane < NR)
            acp = [pltpu.async_copy(act_hbm.at[fidx_a], arow_a, sem3), None]
            for g in range(N_ACT // NR):
                if g + 1 < N_ACT // NR:
                    nxt = fidxs[(g + 1) % 2]
                    plsc.store_scatter(nxt, [lane],
                                       (fbase + (g + 1) * NR) + lane,
                                       mask=lane < NR)
                    acp[(g + 1) % 2] = pltpu.async_copy(
                        act_hbm.at[nxt], bufs[(g + 1) % 2], sem3)
                acp[g % 2].wait()
                cur = bufs[g % 2]
                for f in range(NR):
                    fv = jnp.full((L,), f, i32)

                    def act_row(kk, _):
                        ii = sidx2[j, pl.ds(kk * L, L)]
                        aout[g * NR + f, pl.ds(kk * L, L)] = (
                            plsc.load_gather(cur, [fv, ii]))
                        return 0

                    lax.fori_loop(0, NB, act_row, 0)

            for c in scopies:
                c.wait()
            for kk in range(NB):
                ii = sidx2[j, pl.ds(kk * L, L)]
                s = pl.ds(kk * L, L)
                tail_stage[0, s] = plsc.load_gather(rew1, [zero, ii])
                tail_stage[1, s] = plsc.load_gather(
                    dn1, [zero, ii]).astype(f32)
                tail_stage[2, s] = plsc.load_gather(
                    tr1, [zero, ii]).astype(f32)
            for c in copies:
                c.wait()

            def patch_chunk(kk, _):
                ii = sidx2[j, pl.ds(kk * L, L)]
                m = (ii == tvec).astype(i32)
                nm = jnp.sum(m)

                @pl.when(nm > 0)
                def _():
                    def per_lane(l, _):
                        ml = jnp.sum(jnp.where(lane == l, m, 0))

                        @pl.when(ml > 0)
                        def _():
                            b = jnp.full((L,), kk * L + l, i32)

                            def cp(base, n, ref):
                                def body(c, _):
                                    cols = c * L + lane
                                    plsc.store_scatter(
                                        ref, [b, cols],
                                        plsc.load_gather(
                                            data2, [jv, base + cols]))
                                    return 0
                                lax.fori_loop(0, n // L, body, 0)

                            cp(0, N_OBS, obs_stage)
                            cp(N_OBS + N_ACT, N_OBS, nobs_stage)
                            for c in range(N_ACT // L):
                                cols = c * L + lane
                                plsc.store_scatter(
                                    aout, [cols, b],
                                    plsc.load_gather(
                                        data2, [jv, N_OBS + cols]))
                            c0 = N_OBS + N_ACT + N_OBS
                            for t in range(3):
                                plsc.store_scatter(
                                    tail_stage, [zero + t, b],
                                    plsc.load_gather(
                                        data2,
                                        [jv, jnp.full((L,), c0 + t, i32)]),
                                    mask=lane == 0)
                        return 0

                    lax.fori_loop(0, L, per_lane, 0)
                return 0

            lax.fori_loop(0, NB, patch_chunk, 0)

            el = e - half * ne
            orow = pl.ds(el * BATCH, BATCH)
            outcp = [
                pltpu.async_copy(obs_stage, o_obs.at[orow], sem4),
                pltpu.async_copy(nobs_stage, o_nobs.at[orow], sem4),
                pltpu.async_copy(tail_stage, o_tail.at[:, orow], sem4),
                pltpu.async_copy(aout, o_act.at[pl.ds(el * N_ACT, N_ACT)],
                                 sem4),
            ]
        for c in outcp:
            c.wait()

    return k


ASM_ENVS = 8


def _assemble_kernel(obs_ref, act_ref, nobs_ref, tail_ref, out_ref):
    out_ref[0:N_OBS, :] = obs_ref[...].T
    for g in range(ASM_ENVS):
        cols = pl.ds(g * BATCH, BATCH)
        out_ref[N_OBS:N_OBS + N_ACT, cols] = (
            act_ref[pl.ds(g * N_ACT, N_ACT), :])
    out_ref[N_OBS + N_ACT:2 * N_OBS + N_ACT, :] = nobs_ref[...].T
    out_ref[2 * N_OBS + N_ACT:OUT_D, :] = tail_ref[0:3, :]


def _assemble_kernel_b(obs_ref, act_ref, nobs_ref, tail_ref, init_ref,
                       out_ref):
    del init_ref
    _assemble_kernel(obs_ref, act_ref, nobs_ref, tail_ref, out_ref)


def kernel(observations, actions, rewards, dones, truncations,
           next_observations, obs_data, act_data, next_obs_data, rewards_data,
           dones_data, truncations_data, ptr, sample_idx):
    info = plsc.get_sparse_core_info()
    sck_a = _build_sc_kernel(info.num_cores, info.num_subcores, 0)
    sck_b = _build_sc_kernel(info.num_cores, info.num_subcores, 1)
    t = jnp.asarray(ptr, jnp.int32) % BUF
    tvec = jnp.full((L,), t, jnp.int32)
    data_comb = jnp.concatenate([
        obs_data, act_data, next_obs_data,
        rewards_data[:, None],
        dones_data[:, None].astype(jnp.float32),
        truncations_data[:, None].astype(jnp.float32),
        jnp.zeros((N_ENV, DROW_PAD - OUT_D), jnp.float32),
    ], axis=1)
    sidx = sample_idx.astype(jnp.int32)

    act_t = actions.transpose(0, 2, 1).reshape(N_ENV * N_ACT, BUF)

    sc_args = (
        observations.reshape(N_ENV * BUF, N_OBS),
        next_observations.reshape(N_ENV * BUF, N_OBS),
        act_t, rewards, dones, truncations, data_comb, tvec, sidx)
    a_obs, a_nobs, a_tail, a_act = sck_a(*sc_args)
    b_obs, b_nobs, b_tail, b_act = sck_b(*sc_args)

    nblk = (N_ENV // 2) // ASM_ENVS
    in_specs = [
        pl.BlockSpec((ASM_ENVS * BATCH, N_OBS), lambda i: (i, 0)),
        pl.BlockSpec((ASM_ENVS * N_ACT, BATCH), lambda i: (i, 0)),
        pl.BlockSpec((ASM_ENVS * BATCH, N_OBS), lambda i: (i, 0)),
        pl.BlockSpec((8, ASM_ENVS * BATCH), lambda i: (0, i)),
    ]
    out_shape = jax.ShapeDtypeStruct((OUT_D, N_ENV * BATCH), jnp.float32)
    out_a = pl.pallas_call(
        _assemble_kernel,
        grid=(nblk,),
        in_specs=in_specs,
        out_specs=pl.BlockSpec((OUT_D, ASM_ENVS * BATCH), lambda i: (0, i)),
        out_shape=out_shape,
    )(a_obs, a_act, a_nobs, a_tail)
    out_t = pl.pallas_call(
        _assemble_kernel_b,
        grid=(nblk,),
        in_specs=in_specs + [pl.BlockSpec(memory_space=pltpu.MemorySpace.HBM)],
        out_specs=pl.BlockSpec((OUT_D, ASM_ENVS * BATCH),
                               lambda i: (0, i + nblk)),
        out_shape=out_shape,
        input_output_aliases={4: 0},
    )(b_obs, b_act, b_nobs, b_tail, out_a)
    return out_t.T


# speedup vs baseline: 1.0204x; 1.0204x over previous
import functools

import jax
import jax.numpy as jnp
from jax import lax
from jax.experimental import pallas as pl
from jax.experimental.pallas import tpu as pltpu
from jax.experimental.pallas import tpu_sc as plsc

N_ENV = 64
BUF = 4096
N_OBS = 128
N_ACT = 32
BATCH = 256
OUT_D = N_OBS + N_ACT + N_OBS + 3
DROW_PAD = 384
L = 16
NB = BATCH // L


def _build_sc_kernel(num_cores, num_subcores):
    n_workers = num_cores * num_subcores
    epw = N_ENV // n_workers
    mesh = plsc.VectorSubcoreMesh(core_axis_name="c", subcore_axis_name="s")
    f32 = jnp.float32
    i32 = jnp.int32

    @functools.partial(
        pl.kernel,
        out_type=[
            jax.ShapeDtypeStruct((N_ENV * BATCH, N_OBS), f32),
            jax.ShapeDtypeStruct((N_ENV * BATCH, N_OBS), f32),
            jax.ShapeDtypeStruct((8, N_ENV * BATCH), f32),
            jax.ShapeDtypeStruct((N_ENV * N_ACT, BATCH), f32),
        ],
        mesh=mesh,
        compiler_params=pltpu.CompilerParams(needs_layout_passes=False),
        scratch_types=[
            pltpu.VMEM((epw,), i32),
            pltpu.VMEM((1,), i32),
            pltpu.VMEM((4,), i32),
            pltpu.VMEM((4,), i32),
            pltpu.VMEM((epw, BATCH), i32),
            pltpu.VMEM((1, BUF), f32),
            pltpu.VMEM((1, BUF), i32),
            pltpu.VMEM((1, BUF), i32),
            pltpu.VMEM((epw, DROW_PAD), f32),
            pltpu.VMEM((128,), i32),
            pltpu.VMEM((128,), i32),
            pltpu.VMEM((BATCH, N_OBS), f32),
            pltpu.VMEM((BATCH, N_OBS), f32),
            pltpu.VMEM((8, BATCH), f32),
            pltpu.VMEM((4, BUF), f32),
            pltpu.VMEM((4, BUF), f32),
            pltpu.VMEM((N_ACT, BATCH), f32),
            pltpu.VMEM((L,), i32),
            pltpu.SemaphoreType.DMA,
            pltpu.SemaphoreType.DMA,
            pltpu.SemaphoreType.DMA,
            pltpu.SemaphoreType.DMA,
        ],
    )
    def k(obs_hbm, nobs_hbm, act_hbm, rew_hbm, dn_hbm, tr_hbm, data_hbm,
          tvec_hbm, sidx_hbm,
          o_obs, o_nobs, o_tail, o_act,
          eidx, eidx1, fidx_a, fidx_b, sidx2, rew1, dn1, tr1, data2,
          gidx_a, gidx_b, obs_stage, nobs_stage, tail_stage,
          arow_a, arow_b, aout, tv, sem, sem2, sem3, sem4):
        wid = lax.axis_index("s") * num_cores + lax.axis_index("c")
        lane = lax.iota(i32, L)
        zero = jnp.full((L,), 0, i32)
        e0 = wid * epw
        plsc.store_scatter(eidx, [lane], e0 + lane, mask=lane < epw)
        stage = [
            pltpu.async_copy(sidx_hbm.at[eidx], sidx2, sem),
            pltpu.async_copy(data_hbm.at[eidx], data2, sem),
        ]
        pltpu.sync_copy(tvec_hbm, tv)
        tvec = tv[...]
        for c in stage:
            c.wait()

        outcp = []
        for j in range(epw):
            e = e0 + j
            ebase = e * BUF
            plsc.store_scatter(eidx1, [lane], (e0 + j) + zero, mask=lane < 1)
            scopies = [
                pltpu.async_copy(rew_hbm.at[eidx1], rew1, sem2),
                pltpu.async_copy(dn_hbm.at[eidx1], dn1, sem2),
                pltpu.async_copy(tr_hbm.at[eidx1], tr1, sem2),
            ]
            jv = jnp.full((L,), j, i32)

            for kk in range(NB // 2):
                s = pl.ds(kk * L, L)
                gidx_a[s] = sidx2[j, s] + ebase
            for kk in range(NB // 2):
                s = pl.ds(kk * L, L)
                gidx_b[s] = sidx2[j, pl.ds(128 + kk * L, L)] + ebase
            for c in outcp:
                c.wait()
            outcp = []
            copies = []
            for h, gi in enumerate((gidx_a, gidx_b)):
                rows = pl.ds(h * 128, 128)
                copies.append(pltpu.async_copy(
                    obs_hbm.at[gi], obs_stage.at[rows], sem))
                copies.append(pltpu.async_copy(
                    nobs_hbm.at[gi], nobs_stage.at[rows], sem))

            NR = 4
            fbase = e * N_ACT
            bufs = (arow_a, arow_b)
            fidxs = (fidx_a, fidx_b)
            plsc.store_scatter(fidx_a, [lane], fbase + lane, mask=lane < NR)
            acp = [pltpu.async_copy(act_hbm.at[fidx_a], arow_a, sem3), None]
            for g in range(N_ACT // NR):
                if g + 1 < N_ACT // NR:
                    nxt = fidxs[(g + 1) % 2]
                    plsc.store_scatter(nxt, [lane],
                                       (fbase + (g + 1) * NR) + lane,
                                       mask=lane < NR)
                    acp[(g + 1) % 2] = pltpu.async_copy(
                        act_hbm.at[nxt], bufs[(g + 1) % 2], sem3)
                acp[g % 2].wait()
                cur = bufs[g % 2]
                for f in range(NR):
                    fv = jnp.full((L,), f, i32)

                    def act_row(kk, _):
                        ii = sidx2[j, pl.ds(kk * L, L)]
                        aout[g * NR + f, pl.ds(kk * L, L)] = (
                            plsc.load_gather(cur, [fv, ii]))
                        return 0

                    lax.fori_loop(0, NB, act_row, 0)

            for c in scopies:
                c.wait()
            for kk in range(NB):
                ii = sidx2[j, pl.ds(kk * L, L)]
                s = pl.ds(kk * L, L)
                tail_stage[0, s] = plsc.load_gather(rew1, [zero, ii])
                tail_stage[1, s] = plsc.load_gather(
                    dn1, [zero, ii]).astype(f32)
                tail_stage[2, s] = plsc.load_gather(
                    tr1, [zero, ii]).astype(f32)
            for c in copies:
                c.wait()

            def patch_chunk(kk, _):
                ii = sidx2[j, pl.ds(kk * L, L)]
                m = (ii == tvec).astype(i32)
                nm = jnp.sum(m)

                @pl.when(nm > 0)
                def _():
                    def per_lane(l, _):
                        ml = jnp.sum(jnp.where(lane == l, m, 0))

                        @pl.when(ml > 0)
                        def _():
                            b = jnp.full((L,), kk * L + l, i32)

                            def cp(base, n, ref):
                                def body(c, _):
                                    cols = c * L + lane
                                    plsc.store_scatter(
                                        ref, [b, cols],
                                        plsc.load_gather(
                                            data2, [jv, base + cols]))
                                    return 0
                                lax.fori_loop(0, n // L, body, 0)

                            cp(0, N_OBS, obs_stage)
                            cp(N_OBS + N_ACT, N_OBS, nobs_stage)
                            for c in range(N_ACT // L):
                                cols = c * L + lane
                                plsc.store_scatter(
                                    aout, [cols, b],
                                    plsc.load_gather(
                                        data2, [jv, N_OBS + cols]))
                            c0 = N_OBS + N_ACT + N_OBS
                            for t in range(3):
                                plsc.store_scatter(
                                    tail_stage, [zero + t, b],
                                    plsc.load_gather(
                                        data2,
                                        [jv, jnp.full((L,), c0 + t, i32)]),
                                    mask=lane == 0)
                        return 0

                    lax.fori_loop(0, L, per_lane, 0)
                return 0

            lax.fori_loop(0, NB, patch_chunk, 0)

            orow = pl.ds(e * BATCH, BATCH)
            outcp = [
                pltpu.async_copy(obs_stage, o_obs.at[orow], sem4),
                pltpu.async_copy(nobs_stage, o_nobs.at[orow], sem4),
                pltpu.async_copy(tail_stage, o_tail.at[:, orow], sem4),
                pltpu.async_copy(aout, o_act.at[pl.ds(fbase, N_ACT)], sem4),
            ]
        for c in outcp:
            c.wait()

    return k


ASM_ENVS = 8


def _assemble_kernel(obs_ref, act_ref, nobs_ref, tail_ref, out_ref):
    out_ref[0:N_OBS, :] = obs_ref[...].T
    for g in range(ASM_ENVS):
        cols = pl.ds(g * BATCH, BATCH)
        out_ref[N_OBS:N_OBS + N_ACT, cols] = (
            act_ref[pl.ds(g * N_ACT, N_ACT), :])
    out_ref[N_OBS + N_ACT:2 * N_OBS + N_ACT, :] = nobs_ref[...].T
    out_ref[2 * N_OBS + N_ACT:OUT_D, :] = tail_ref[0:3, :]


def kernel(observations, actions, rewards, dones, truncations,
           next_observations, obs_data, act_data, next_obs_data, rewards_data,
           dones_data, truncations_data, ptr, sample_idx):
    info = plsc.get_sparse_core_info()
    sck = _build_sc_kernel(info.num_cores, info.num_subcores)
    t = jnp.asarray(ptr, jnp.int32) % BUF
    tvec = jnp.full((L,), t, jnp.int32)
    data_comb = jnp.concatenate([
        obs_data, act_data, next_obs_data,
        rewards_data[:, None],
        dones_data[:, None].astype(jnp.float32),
        truncations_data[:, None].astype(jnp.float32),
        jnp.zeros((N_ENV, DROW_PAD - OUT_D), jnp.float32),
    ], axis=1)
    sidx = sample_idx.astype(jnp.int32)

    act_t = actions.transpose(0, 2, 1).reshape(N_ENV * N_ACT, BUF)

    s_obs, s_nobs, s_tail, s_act_t = sck(
        observations.reshape(N_ENV * BUF, N_OBS),
        next_observations.reshape(N_ENV * BUF, N_OBS),
        act_t, rewards, dones, truncations, data_comb, tvec, sidx)

    out_t = pl.pallas_call(
        _assemble_kernel,
        grid=(N_ENV // ASM_ENVS,),
        in_specs=[
            pl.BlockSpec((ASM_ENVS * BATCH, N_OBS), lambda i: (i, 0)),
            pl.BlockSpec((ASM_ENVS * N_ACT, BATCH), lambda i: (i, 0)),
            pl.BlockSpec((ASM_ENVS * BATCH, N_OBS), lambda i: (i, 0)),
            pl.BlockSpec((8, ASM_ENVS * BATCH), lambda i: (0, i)),
        ],
        out_specs=pl.BlockSpec((OUT_D, ASM_ENVS * BATCH), lambda i: (0, i)),
        out_shape=jax.ShapeDtypeStruct((OUT_D, N_ENV * BATCH), jnp.float32),
    )(s_obs, s_act_t, s_nobs, s_tail)
    return out_t.T
